# baseline (speedup 1.0000x reference)
import functools

import jax
import jax.numpy as jnp
from jax import lax
from jax.experimental import pallas as pl
from jax.experimental.pallas import tpu as pltpu
from jax.experimental.pallas import tpu_sc as plsc

N_NODES = 50000
E_EDGES = 800000
NC, NS = 2, 16
N_PAD = 50048
E_PAD = 819200
ROWS = E_PAD // 128
RPT = N_PAD // NS

_mesh = plsc.VectorSubcoreMesh(
    core_axis_name="c", subcore_axis_name="s", num_cores=NC, num_subcores=NS)
_sc_params = pltpu.CompilerParams(
    needs_layout_passes=False, use_tc_tiling_on_sc=False)


_vec_out = tuple(
    jax.ShapeDtypeStruct((NC * N_PAD,), jnp.float32) for _ in range(4))


@functools.partial(
    pl.kernel,
    out_type=_vec_out,
    mesh=_mesh,
    scratch_types=[
        pltpu.VMEM((2 * N_NODES,), jnp.float32),
        pltpu.VMEM((4, 128), jnp.int32),
        pltpu.VMEM((4, 128), jnp.int32),
        pltpu.VMEM((4, 128), jnp.float32),
        pltpu.VMEM((4, 128), jnp.float32),
        pltpu.VMEM((4, 128), jnp.float32),
        pltpu.VMEM((4, 128), jnp.float32),
        pltpu.VMEM((4, 128), jnp.float32),
        pltpu.VMEM((1024,), jnp.float32),
        pltpu.VMEM((RPT,), jnp.float32),
        pltpu.VMEM_SHARED((N_PAD,), jnp.float32),
        pltpu.VMEM_SHARED((N_PAD,), jnp.float32),
        pltpu.VMEM_SHARED((N_PAD,), jnp.float32),
        pltpu.VMEM_SHARED((N_PAD,), jnp.float32),
        pltpu.SemaphoreType.DMA,
    ],
    compiler_params=_sc_params,
)
def _edge_pass1(src_hbm, dst_hbm, w_hbm, x_hbm,
                outP, outM, outX0, outX1,
                x_v, sidx, didx, wbuf, pbuf, mbuf, x0b, x1b, zv, obuf,
                accP, accM, accX0, accX1, ssem):
    c = lax.axis_index("c")
    s = lax.axis_index("s")
    wid = c * NS + s
    row0 = s * RPT
    pltpu.sync_copy(x_hbm, x_v)
    fz = jnp.zeros((16,), jnp.float32)
    for i in range(64):
        zv[pl.ds(i * 16, 16)] = fz
    for acc in (accP, accM, accX0, accX1):
        for k in range(3):
            pltpu.sync_copy(zv, acc.at[pl.ds(row0 + k * 1024, 1024)])
        pltpu.sync_copy(zv.at[pl.ds(0, RPT - 3072)],
                        acc.at[pl.ds(row0 + 3072, RPT - 3072)])
    plsc.subcore_barrier()

    nblocks = (ROWS // 32) // 4
    base_row = wid * (ROWS // 32)

    def scatters(didx_ref):
        return [pltpu.make_async_copy(vb.at[r], acc.at[didx_ref.at[r]], ssem)
                for r in range(4)
                for vb, acc in ((pbuf, accP), (mbuf, accM),
                                (x0b, accX0), (x1b, accX1))]

    def body(b, carry):
        r0 = base_row + b * 4
        pltpu.sync_copy(src_hbm.at[pl.ds(r0, 4)], sidx)
        pltpu.sync_copy(dst_hbm.at[pl.ds(r0, 4)], didx)
        pltpu.sync_copy(w_hbm.at[pl.ds(r0, 4)], wbuf)
        for r in range(4):
            for g in range(8):
                w16 = wbuf[r, pl.ds(g * 16, 16)]
                s16 = sidx[r, pl.ds(g * 16, 16)]
                pbuf[r, pl.ds(g * 16, 16)] = jnp.maximum(w16, 0.0)
                mbuf[r, pl.ds(g * 16, 16)] = jnp.maximum(-w16, 0.0)
                x0b[r, pl.ds(g * 16, 16)] = plsc.load_gather(x_v, [s16 * 2])
                x1b[r, pl.ds(g * 16, 16)] = plsc.load_gather(x_v, [s16 * 2 + 1])
        ds = scatters(didx)
        for d in ds:
            d.start(add=True)
        for d in ds:
            d.wait()
        return carry

    lax.fori_loop(0, nblocks, body, 0)
    plsc.subcore_barrier()
    off = pl.multiple_of(c * N_PAD + row0, 8)
    for acc, out in ((accP, outP), (accM, outM), (accX0, outX0), (accX1, outX1)):
        pltpu.sync_copy(acc.at[pl.ds(row0, RPT)], obuf)
        pltpu.sync_copy(obuf, out.at[pl.ds(off, RPT)])


@functools.partial(
    pl.kernel,
    out_type=jax.ShapeDtypeStruct((NC, N_PAD, 32), jnp.float32),
    mesh=_mesh,
    scratch_types=[
        pltpu.VMEM((8, 128), jnp.int32),
        pltpu.VMEM((8, 128), jnp.int32),
        pltpu.VMEM((8, 128), jnp.int32),
        pltpu.VMEM((128, 32), jnp.float32),
        pltpu.VMEM((128, 32), jnp.float32),
        pltpu.VMEM_SHARED((N_PAD, 32), jnp.float32),
        pltpu.SemaphoreType.DMA,
        pltpu.SemaphoreType.DMA,
        pltpu.SemaphoreType.DMA,
        pltpu.SemaphoreType.DMA,
    ],
    compiler_params=_sc_params,
)
def _edge_pass2(src_hbm, dst_hbm, h1_hbm, out_hbm,
                sidx, gidx, didx, gbufA, gbufB, acc,
                gsemA, gsemB, ssemA, ssemB):
    c = lax.axis_index("c")
    s = lax.axis_index("s")
    row0 = s * RPT

    fz = jnp.zeros((16,), jnp.float32)
    for i in range(128):
        gbufA[i, pl.ds(0, 16)] = fz
        gbufA[i, pl.ds(16, 16)] = fz
    for k in range(24):
        pltpu.sync_copy(gbufA, acc.at[pl.ds(row0 + k * 128, 128)])
    pltpu.sync_copy(gbufA.at[pl.ds(0, RPT - 3072)],
                    acc.at[pl.ds(row0 + 3072, RPT - 3072)])
    plsc.subcore_barrier()

    rows_per_tile = ROWS // NS
    nblocks = rows_per_tile // 8
    base_row = s * rows_per_tile
    gbufs = (gbufA, gbufB)
    gsems = (gsemA, gsemB)
    ssems = (ssemA, ssemB)

    def drain(j):
        pltpu.make_async_copy(
            gbufs[j % 2], acc.at[didx.at[j]], ssems[j % 2]).wait()

    def finish_gather_fire_scatter(j):
        gbuf = gbufs[j % 2]
        pltpu.make_async_copy(h1_hbm.at[gidx.at[j]], gbuf, gsems[j % 2]).wait()
        pltpu.make_async_copy(
            gbuf, acc.at[didx.at[j]], ssems[j % 2]).start(add=True)

    def body(b, carry):
        @pl.when(b > 0)
        def _():
            drain(6)
            drain(7)
        r0 = base_row + b * 8
        pltpu.sync_copy(src_hbm.at[pl.ds(r0, 8)], sidx)
        pltpu.sync_copy(dst_hbm.at[pl.ds(r0, 8)], didx)
        for j in range(8):
            for g in range(8):
                s16 = sidx[j, pl.ds(g * 16, 16)]
                gidx[j, pl.ds(g * 16, 16)] = s16 * 2 + c
        for j in range(8):
            gbuf, gs = gbufs[j % 2], gsems[j % 2]
            if j >= 2:
                drain(j - 2)
            pltpu.make_async_copy(h1_hbm.at[gidx.at[j]], gbuf, gs).start()
            if j >= 1:
                finish_gather_fire_scatter(j - 1)
        finish_gather_fire_scatter(7)
        return carry

    lax.fori_loop(0, nblocks, body, 0)
    drain(6)
    drain(7)
    plsc.subcore_barrier()

    for k in range(24):
        pltpu.sync_copy(acc.at[pl.ds(row0 + k * 128, 128)], gbufA)
        pltpu.sync_copy(gbufA, out_hbm.at[c, pl.ds(row0 + k * 128, 128)])
    rem = RPT - 3072
    pltpu.sync_copy(acc.at[pl.ds(row0 + 3072, rem)], gbufA.at[pl.ds(0, rem)])
    pltpu.sync_copy(gbufA.at[pl.ds(0, rem)],
                    out_hbm.at[c, pl.ds(row0 + 3072, rem)])


_R = 2000


def _split_acc(a):
    p = a[:, 0:1] + a[:, 1:2]
    m = a[:, 2:3] + a[:, 3:4]
    hf = jnp.concatenate(
        [a[:, 4:5] + a[:, 5:6], a[:, 6:7] + a[:, 7:8]], axis=1)
    return p, m, hf


def _dense1_body(x_ref, acc_ref, Wx_ref, Wf_ref, bf_ref, wvec_ref, Ww_ref,
                 out_ref):
    p, m, hf = _split_acc(acc_ref[...])
    wv = wvec_ref[...]
    u = jnp.sum(jnp.maximum(wv, 0.0) * Ww_ref[...], axis=0, keepdims=True)
    v = jnp.sum(jnp.maximum(-wv, 0.0) * Ww_ref[...], axis=0, keepdims=True)
    h = (jnp.dot(x_ref[...], Wx_ref[...], preferred_element_type=jnp.float32)
         + jnp.dot(hf, Wf_ref[...], preferred_element_type=jnp.float32)
         + p * u + m * v + bf_ref[...])
    out_ref[...] = jnp.maximum(h, 0.0)


def _dense2_body(x_ref, acc_ref, hf2_ref, Wx_ref, Wf_ref, bf_ref, wvec_ref,
                 Ww_ref, out_ref):
    p, m, _ = _split_acc(acc_ref[...])
    wv = wvec_ref[...]
    u = jnp.sum(jnp.maximum(wv, 0.0) * Ww_ref[...], axis=0, keepdims=True)
    v = jnp.sum(jnp.maximum(-wv, 0.0) * Ww_ref[...], axis=0, keepdims=True)
    Wf = Wf_ref[...]
    h = (jnp.dot(x_ref[...], Wx_ref[...], preferred_element_type=jnp.float32)
         + jnp.dot(hf2_ref[0], Wf[:32, :], preferred_element_type=jnp.float32)
         + jnp.dot(hf2_ref[1], Wf[32:, :], preferred_element_type=jnp.float32)
         + p * u + m * v + bf_ref[...])
    out_ref[...] = jnp.maximum(h, 0.0)


def _dense1(x, acc, Wx, Wf, bf, wvec, Ww):
    grid = N_NODES // _R
    return pl.pallas_call(
        _dense1_body,
        grid=(grid,),
        in_specs=[
            pl.BlockSpec((_R, 2), lambda i: (i, 0)),
            pl.BlockSpec((_R, 8), lambda i: (i, 0)),
            pl.BlockSpec((2, 64), lambda i: (0, 0)),
            pl.BlockSpec((2, 64), lambda i: (0, 0)),
            pl.BlockSpec((1, 64), lambda i: (0, 0)),
            pl.BlockSpec((64, 1), lambda i: (0, 0)),
            pl.BlockSpec((64, 64), lambda i: (0, 0)),
        ],
        out_specs=pl.BlockSpec((_R, 64), lambda i: (i, 0)),
        out_shape=jax.ShapeDtypeStruct((N_NODES, 64), jnp.float32),
    )(x, acc, Wx, Wf, bf, wvec, Ww)


def _dense2(x, acc, hf2, Wx, Wf, bf, wvec, Ww):
    grid = N_NODES // _R
    return pl.pallas_call(
        _dense2_body,
        grid=(grid,),
        in_specs=[
            pl.BlockSpec((_R, 2), lambda i: (i, 0)),
            pl.BlockSpec((_R, 8), lambda i: (i, 0)),
            pl.BlockSpec((2, _R, 32), lambda i: (0, i, 0)),
            pl.BlockSpec((2, 64), lambda i: (0, 0)),
            pl.BlockSpec((64, 64), lambda i: (0, 0)),
            pl.BlockSpec((1, 64), lambda i: (0, 0)),
            pl.BlockSpec((64, 1), lambda i: (0, 0)),
            pl.BlockSpec((64, 64), lambda i: (0, 0)),
        ],
        out_specs=pl.BlockSpec((_R, 64), lambda i: (i, 0)),
        out_shape=jax.ShapeDtypeStruct((N_NODES, 64), jnp.float32),
    )(x, acc, hf2, Wx, Wf, bf, wvec, Ww)


def kernel(x, edge_index, edge_w, Wx1, Ww1, Wf1, bf1, wvec1,
           Wx2, Ww2, Wf2, bf2, wvec2):
    src = edge_index[0].astype(jnp.int32)
    dst = edge_index[1].astype(jnp.int32)
    pad = E_PAD - E_EDGES
    pad_dst = N_NODES + (jnp.arange(pad, dtype=jnp.int32) % (N_PAD - N_NODES))
    src_p = jnp.concatenate(
        [src, jnp.zeros((pad,), jnp.int32)]).reshape(ROWS, 128)
    dst_p = jnp.concatenate([dst, pad_dst]).reshape(ROWS, 128)
    w_p = jnp.concatenate(
        [edge_w.astype(jnp.float32), jnp.zeros((pad,), jnp.float32)]
    ).reshape(ROWS, 128)
    x_flat = x.reshape(-1)

    accP, accM, accX0, accX1 = (
        a.reshape(NC, N_PAD) for a in _edge_pass1(src_p, dst_p, w_p, x_flat))
    acc8 = jnp.stack(
        [accP[0], accP[1], accM[0], accM[1],
         accX0[0], accX0[1], accX1[0], accX1[1]], axis=1)
    h1 = _dense1(x, acc8, Wx1, Wf1, bf1.reshape(1, 64),
                 wvec1.reshape(64, 1), Ww1)
    hf2 = _edge_pass2(src_p, dst_p, h1.reshape(2 * N_NODES, 32))
    out = _dense2(x, acc8, hf2, Wx2, Wf2, bf2.reshape(1, 64),
                  wvec2.reshape(64, 1), Ww2)
    return out


# speedup vs baseline: 1.0209x; 1.0209x over previous
import functools

import jax
import jax.numpy as jnp
from jax import lax
from jax.experimental import pallas as pl
from jax.experimental.pallas import tpu as pltpu
from jax.experimental.pallas import tpu_sc as plsc

N_NODES = 50000
E_EDGES = 800000
NC, NS = 2, 16
N_PAD = 50048
E_PAD = 819200
ROWS = E_PAD // 128
RPT = N_PAD // NS

_mesh = plsc.VectorSubcoreMesh(
    core_axis_name="c", subcore_axis_name="s", num_cores=NC, num_subcores=NS)
_sc_params = pltpu.CompilerParams(
    needs_layout_passes=False, use_tc_tiling_on_sc=False)


_vec_out = tuple(
    jax.ShapeDtypeStruct((NC * N_PAD,), jnp.float32) for _ in range(4))


@functools.partial(
    pl.kernel,
    out_type=_vec_out,
    mesh=_mesh,
    scratch_types=[
        pltpu.VMEM((2 * N_NODES,), jnp.float32),
        pltpu.VMEM((4, 128), jnp.int32),
        pltpu.VMEM((4, 128), jnp.int32),
        pltpu.VMEM((4, 128), jnp.float32),
        pltpu.VMEM((4, 128), jnp.float32),
        pltpu.VMEM((4, 128), jnp.float32),
        pltpu.VMEM((4, 128), jnp.float32),
        pltpu.VMEM((4, 128), jnp.float32),
        pltpu.VMEM((1024,), jnp.float32),
        pltpu.VMEM((RPT,), jnp.float32),
        pltpu.VMEM_SHARED((N_PAD,), jnp.float32),
        pltpu.VMEM_SHARED((N_PAD,), jnp.float32),
        pltpu.VMEM_SHARED((N_PAD,), jnp.float32),
        pltpu.VMEM_SHARED((N_PAD,), jnp.float32),
        pltpu.SemaphoreType.DMA,
    ],
    compiler_params=_sc_params,
)
def _edge_pass1(src_hbm, dst_hbm, w_hbm, x_hbm,
                outP, outM, outX0, outX1,
                x_v, sidx, didx, wbuf, pbuf, mbuf, x0b, x1b, zv, obuf,
                accP, accM, accX0, accX1, ssem):
    c = lax.axis_index("c")
    s = lax.axis_index("s")
    wid = c * NS + s
    row0 = s * RPT
    pltpu.sync_copy(x_hbm, x_v)
    fz = jnp.zeros((16,), jnp.float32)
    for i in range(64):
        zv[pl.ds(i * 16, 16)] = fz
    for acc in (accP, accM, accX0, accX1):
        for k in range(3):
            pltpu.sync_copy(zv, acc.at[pl.ds(row0 + k * 1024, 1024)])
        pltpu.sync_copy(zv.at[pl.ds(0, RPT - 3072)],
                        acc.at[pl.ds(row0 + 3072, RPT - 3072)])
    plsc.subcore_barrier()

    nblocks = (ROWS // 32) // 4
    base_row = wid * (ROWS // 32)

    def scatters(didx_ref):
        return [pltpu.make_async_copy(vb.at[r], acc.at[didx_ref.at[r]], ssem)
                for r in range(4)
                for vb, acc in ((pbuf, accP), (mbuf, accM),
                                (x0b, accX0), (x1b, accX1))]

    def body(b, carry):
        r0 = base_row + b * 4
        pltpu.sync_copy(src_hbm.at[pl.ds(r0, 4)], sidx)
        pltpu.sync_copy(dst_hbm.at[pl.ds(r0, 4)], didx)
        pltpu.sync_copy(w_hbm.at[pl.ds(r0, 4)], wbuf)
        for r in range(4):
            for g in range(8):
                w16 = wbuf[r, pl.ds(g * 16, 16)]
                s16 = sidx[r, pl.ds(g * 16, 16)]
                pbuf[r, pl.ds(g * 16, 16)] = jnp.maximum(w16, 0.0)
                mbuf[r, pl.ds(g * 16, 16)] = jnp.maximum(-w16, 0.0)
                x0b[r, pl.ds(g * 16, 16)] = plsc.load_gather(x_v, [s16 * 2])
                x1b[r, pl.ds(g * 16, 16)] = plsc.load_gather(x_v, [s16 * 2 + 1])
        ds = scatters(didx)
        for d in ds:
            d.start(add=True)
        for d in ds:
            d.wait()
        return carry

    lax.fori_loop(0, nblocks, body, 0)
    plsc.subcore_barrier()
    off = pl.multiple_of(c * N_PAD + row0, 8)
    for acc, out in ((accP, outP), (accM, outM), (accX0, outX0), (accX1, outX1)):
        pltpu.sync_copy(acc.at[pl.ds(row0, RPT)], obuf)
        pltpu.sync_copy(obuf, out.at[pl.ds(off, RPT)])


_NB = 10
_W = 5
_QR = 100


@functools.partial(
    pl.kernel,
    out_type=jax.ShapeDtypeStruct((4, N_PAD, 16), jnp.float32),
    mesh=_mesh,
    scratch_types=(
        [pltpu.VMEM((_QR, 128), jnp.int32),
         pltpu.VMEM((_QR, 128), jnp.int32)]
        + [pltpu.VMEM((128, 16), jnp.float32)] * _NB
        + [pltpu.VMEM_SHARED((N_PAD, 16), jnp.float32)]
        + [pltpu.SemaphoreType.DMA] * (2 * _NB)
    ),
    compiler_params=_sc_params,
)
def _edge_pass2(src_hbm, dst_hbm, h1_hbm, out_hbm, sq, dq, *rest):
    ring = rest[:_NB]
    acc = rest[_NB]
    gsems = rest[_NB + 1:2 * _NB + 1]
    ssems = rest[2 * _NB + 1:]
    c = lax.axis_index("c")
    s = lax.axis_index("s")
    row0 = s * RPT
    rows_per_tile = ROWS // NS
    base_row = s * rows_per_tile

    fz = jnp.zeros((16,), jnp.float32)
    for i in range(128):
        ring[0][i, pl.ds(0, 16)] = fz

    def finish(row, k):
        pltpu.make_async_copy(h1_hbm.at[sq.at[row]], ring[k], gsems[k]).wait()
        pltpu.make_async_copy(
            ring[k], acc.at[dq.at[row]], ssems[k]).start(add=True)

    def drain(row, k):
        pltpu.make_async_copy(ring[k], acc.at[dq.at[row]], ssems[k]).wait()

    for q in range(2):
        plane = c * 2 + q
        for k in range(24):
            pltpu.sync_copy(ring[0], acc.at[pl.ds(row0 + k * 128, 128)])
        pltpu.sync_copy(ring[0].at[pl.ds(0, RPT - 3072)],
                        acc.at[pl.ds(row0 + 3072, RPT - 3072)])
        plsc.subcore_barrier()

        for quarter in range(rows_per_tile // _QR):
            q0 = base_row + quarter * _QR
            pltpu.sync_copy(src_hbm.at[pl.ds(q0, _QR)], sq)
            pltpu.sync_copy(dst_hbm.at[pl.ds(q0, _QR)], dq)

            def body(it, carry):
                for k in range(_NB):
                    j = it * _NB + k
                    pl.when(it > 0)(functools.partial(drain, j - _NB, k))
                    for g in range(8):
                        s16 = sq[j, pl.ds(g * 16, 16)]
                        sq[j, pl.ds(g * 16, 16)] = s16 * 4 + plane
                    pltpu.make_async_copy(
                        h1_hbm.at[sq.at[j]], ring[k], gsems[k]).start()
                    ko = (k + _W) % _NB
                    if k < _NB - _W:
                        pl.when(it > 0)(
                            functools.partial(finish, j + _W - _NB, ko))
                    else:
                        finish(j - _W, ko)
                return carry

            lax.fori_loop(0, _QR // _NB, body, 0)
            for k in range(_W):
                finish(_QR - _W + k, _NB - _W + k)
            for k in range(_NB):
                drain(_QR - _NB + k, k)

        plsc.subcore_barrier()
        for k in range(24):
            pltpu.sync_copy(acc.at[pl.ds(row0 + k * 128, 128)], ring[1])
            pltpu.sync_copy(
                ring[1], out_hbm.at[plane, pl.ds(row0 + k * 128, 128)])
        rem = RPT - 3072
        pltpu.sync_copy(acc.at[pl.ds(row0 + 3072, rem)],
                        ring[1].at[pl.ds(0, rem)])
        pltpu.sync_copy(ring[1].at[pl.ds(0, rem)],
                        out_hbm.at[plane, pl.ds(row0 + 3072, rem)])
        plsc.subcore_barrier()


_R = 2000


def _split_acc(a):
    p = a[:, 0:1] + a[:, 1:2]
    m = a[:, 2:3] + a[:, 3:4]
    hf = jnp.concatenate(
        [a[:, 4:5] + a[:, 5:6], a[:, 6:7] + a[:, 7:8]], axis=1)
    return p, m, hf


def _dense1_body(x_ref, acc_ref, Wx_ref, Wf_ref, bf_ref, wvec_ref, Ww_ref,
                 out_ref):
    p, m, hf = _split_acc(acc_ref[...])
    wv = wvec_ref[...]
    u = jnp.sum(jnp.maximum(wv, 0.0) * Ww_ref[...], axis=0, keepdims=True)
    v = jnp.sum(jnp.maximum(-wv, 0.0) * Ww_ref[...], axis=0, keepdims=True)
    h = (jnp.dot(x_ref[...], Wx_ref[...], preferred_element_type=jnp.float32)
         + jnp.dot(hf, Wf_ref[...], preferred_element_type=jnp.float32)
         + p * u + m * v + bf_ref[...])
    out_ref[...] = jnp.maximum(h, 0.0)


def _dense2_body(x_ref, acc_ref, hf2_ref, Wx_ref, Wf_ref, bf_ref, wvec_ref,
                 Ww_ref, out_ref):
    p, m, _ = _split_acc(acc_ref[...])
    wv = wvec_ref[...]
    u = jnp.sum(jnp.maximum(wv, 0.0) * Ww_ref[...], axis=0, keepdims=True)
    v = jnp.sum(jnp.maximum(-wv, 0.0) * Ww_ref[...], axis=0, keepdims=True)
    Wf = Wf_ref[...]
    h = (jnp.dot(x_ref[...], Wx_ref[...], preferred_element_type=jnp.float32)
         + p * u + m * v + bf_ref[...])
    for qq in range(4):
        h = h + jnp.dot(hf2_ref[qq], Wf[16 * qq:16 * (qq + 1), :],
                        preferred_element_type=jnp.float32)
    out_ref[...] = jnp.maximum(h, 0.0)


def _dense1(x, acc, Wx, Wf, bf, wvec, Ww):
    grid = N_NODES // _R
    return pl.pallas_call(
        _dense1_body,
        grid=(grid,),
        in_specs=[
            pl.BlockSpec((_R, 2), lambda i: (i, 0)),
            pl.BlockSpec((_R, 8), lambda i: (i, 0)),
            pl.BlockSpec((2, 64), lambda i: (0, 0)),
            pl.BlockSpec((2, 64), lambda i: (0, 0)),
            pl.BlockSpec((1, 64), lambda i: (0, 0)),
            pl.BlockSpec((64, 1), lambda i: (0, 0)),
            pl.BlockSpec((64, 64), lambda i: (0, 0)),
        ],
        out_specs=pl.BlockSpec((_R, 64), lambda i: (i, 0)),
        out_shape=jax.ShapeDtypeStruct((N_NODES, 64), jnp.float32),
    )(x, acc, Wx, Wf, bf, wvec, Ww)


def _dense2(x, acc, hf2, Wx, Wf, bf, wvec, Ww):
    grid = N_NODES // _R
    return pl.pallas_call(
        _dense2_body,
        grid=(grid,),
        in_specs=[
            pl.BlockSpec((_R, 2), lambda i: (i, 0)),
            pl.BlockSpec((_R, 8), lambda i: (i, 0)),
            pl.BlockSpec((4, _R, 16), lambda i: (0, i, 0)),
            pl.BlockSpec((2, 64), lambda i: (0, 0)),
            pl.BlockSpec((64, 64), lambda i: (0, 0)),
            pl.BlockSpec((1, 64), lambda i: (0, 0)),
            pl.BlockSpec((64, 1), lambda i: (0, 0)),
            pl.BlockSpec((64, 64), lambda i: (0, 0)),
        ],
        out_specs=pl.BlockSpec((_R, 64), lambda i: (i, 0)),
        out_shape=jax.ShapeDtypeStruct((N_NODES, 64), jnp.float32),
    )(x, acc, hf2, Wx, Wf, bf, wvec, Ww)


def kernel(x, edge_index, edge_w, Wx1, Ww1, Wf1, bf1, wvec1,
           Wx2, Ww2, Wf2, bf2, wvec2):
    src = edge_index[0].astype(jnp.int32)
    dst = edge_index[1].astype(jnp.int32)
    pad = E_PAD - E_EDGES
    pad_dst = N_NODES + (jnp.arange(pad, dtype=jnp.int32) % (N_PAD - N_NODES))
    src_p = jnp.concatenate(
        [src, jnp.zeros((pad,), jnp.int32)]).reshape(ROWS, 128)
    dst_p = jnp.concatenate([dst, pad_dst]).reshape(ROWS, 128)
    w_p = jnp.concatenate(
        [edge_w.astype(jnp.float32), jnp.zeros((pad,), jnp.float32)]
    ).reshape(ROWS, 128)
    x_flat = x.reshape(-1)

    accP, accM, accX0, accX1 = (
        a.reshape(NC, N_PAD) for a in _edge_pass1(src_p, dst_p, w_p, x_flat))
    acc8 = jnp.stack(
        [accP[0], accP[1], accM[0], accM[1],
         accX0[0], accX0[1], accX1[0], accX1[1]], axis=1)
    h1 = _dense1(x, acc8, Wx1, Wf1, bf1.reshape(1, 64),
                 wvec1.reshape(64, 1), Ww1)
    hf2 = _edge_pass2(src_p, dst_p, h1.reshape(4 * N_NODES, 16))
    out = _dense2(x, acc8, hf2, Wx2, Wf2, bf2.reshape(1, 64),
                  wvec2.reshape(64, 1), Ww2)
    return out


# speedup vs baseline: 1.0831x; 1.0609x over previous
import functools

import jax
import jax.numpy as jnp
from jax import lax
from jax.experimental import pallas as pl
from jax.experimental.pallas import tpu as pltpu
from jax.experimental.pallas import tpu_sc as plsc

N_NODES = 50000
E_EDGES = 800000
NC, NS = 2, 16
N_PAD = 50048
E_PAD = 819200
ROWS = E_PAD // 128
RPT = N_PAD // NS

_mesh = plsc.VectorSubcoreMesh(
    core_axis_name="c", subcore_axis_name="s", num_cores=NC, num_subcores=NS)
_sc_params = pltpu.CompilerParams(
    needs_layout_passes=False, use_tc_tiling_on_sc=False)


_vec_out = tuple(
    jax.ShapeDtypeStruct((NC * N_PAD,), jnp.float32) for _ in range(4))


@functools.partial(
    pl.kernel,
    out_type=_vec_out,
    mesh=_mesh,
    scratch_types=[
        pltpu.VMEM((2 * N_NODES,), jnp.float32),
        pltpu.VMEM((512,), jnp.int32),
        pltpu.VMEM((512,), jnp.int32),
        pltpu.VMEM((512,), jnp.float32),
        pltpu.VMEM((4, 128), jnp.float32),
        pltpu.VMEM((4, 128), jnp.float32),
        pltpu.VMEM((4, 128), jnp.float32),
        pltpu.VMEM((4, 128), jnp.float32),
        pltpu.VMEM((1024,), jnp.float32),
        pltpu.VMEM((RPT,), jnp.float32),
        pltpu.VMEM_SHARED((N_PAD,), jnp.float32),
        pltpu.VMEM_SHARED((N_PAD,), jnp.float32),
        pltpu.VMEM_SHARED((N_PAD,), jnp.float32),
        pltpu.VMEM_SHARED((N_PAD,), jnp.float32),
        pltpu.SemaphoreType.DMA,
    ],
    compiler_params=_sc_params,
)
def _edge_pass1(src_hbm, dst_hbm, w_hbm, x_hbm,
                outP, outM, outX0, outX1,
                x_v, sidx, didx, wbuf, pbuf, mbuf, x0b, x1b, zv, obuf,
                accP, accM, accX0, accX1, ssem):
    c = lax.axis_index("c")
    s = lax.axis_index("s")
    wid = c * NS + s
    row0 = s * RPT
    pltpu.sync_copy(x_hbm, x_v)
    fz = jnp.zeros((16,), jnp.float32)
    for i in range(64):
        zv[pl.ds(i * 16, 16)] = fz
    for acc in (accP, accM, accX0, accX1):
        for k in range(3):
            pltpu.sync_copy(zv, acc.at[pl.ds(row0 + k * 1024, 1024)])
        pltpu.sync_copy(zv.at[pl.ds(0, RPT - 3072)],
                        acc.at[pl.ds(row0 + 3072, RPT - 3072)])
    plsc.subcore_barrier()

    nblocks = (E_PAD // 32) // 512
    base_e = wid * (E_PAD // 32)

    def scatters(didx_ref):
        return [pltpu.make_async_copy(
                    vb.at[r], acc.at[didx_ref.at[pl.ds(r * 128, 128)]], ssem)
                for r in range(4)
                for vb, acc in ((pbuf, accP), (mbuf, accM),
                                (x0b, accX0), (x1b, accX1))]

    def body(b, carry):
        e0 = base_e + b * 512
        pltpu.sync_copy(src_hbm.at[pl.ds(e0, 512)], sidx)
        pltpu.sync_copy(dst_hbm.at[pl.ds(e0, 512)], didx)
        pltpu.sync_copy(w_hbm.at[pl.ds(e0, 512)], wbuf)
        for r in range(4):
            for g in range(8):
                o = r * 128 + g * 16
                w16 = wbuf[pl.ds(o, 16)]
                s16 = sidx[pl.ds(o, 16)]
                pbuf[r, pl.ds(g * 16, 16)] = jnp.maximum(w16, 0.0)
                mbuf[r, pl.ds(g * 16, 16)] = jnp.maximum(-w16, 0.0)
                x0b[r, pl.ds(g * 16, 16)] = plsc.load_gather(x_v, [s16 * 2])
                x1b[r, pl.ds(g * 16, 16)] = plsc.load_gather(x_v, [s16 * 2 + 1])
        ds = scatters(didx)
        for d in ds:
            d.start(add=True)
        for d in ds:
            d.wait()
        return carry

    lax.fori_loop(0, nblocks, body, 0)
    plsc.subcore_barrier()
    off = pl.multiple_of(c * N_PAD + row0, 8)
    for acc, out in ((accP, outP), (accM, outM), (accX0, outX0), (accX1, outX1)):
        pltpu.sync_copy(acc.at[pl.ds(row0, RPT)], obuf)
        pltpu.sync_copy(obuf, out.at[pl.ds(off, RPT)])


@functools.partial(
    pl.kernel,
    out_type=jax.ShapeDtypeStruct((NC, N_PAD, 32), jnp.float32),
    mesh=_mesh,
    scratch_types=[
        pltpu.VMEM((1024,), jnp.int32),
        pltpu.VMEM((1024,), jnp.int32),
        pltpu.VMEM((1024,), jnp.int32),
        pltpu.VMEM((128, 32), jnp.float32),
        pltpu.VMEM((128, 32), jnp.float32),
        pltpu.VMEM_SHARED((N_PAD, 32), jnp.float32),
        pltpu.SemaphoreType.DMA,
        pltpu.SemaphoreType.DMA,
        pltpu.SemaphoreType.DMA,
        pltpu.SemaphoreType.DMA,
    ],
    compiler_params=_sc_params,
)
def _edge_pass2(src_hbm, dst_hbm, h1_hbm, out_hbm,
                sidx, gidx, didx, gbufA, gbufB, acc,
                gsemA, gsemB, ssemA, ssemB):
    c = lax.axis_index("c")
    s = lax.axis_index("s")
    row0 = s * RPT

    fz = jnp.zeros((16,), jnp.float32)
    for i in range(128):
        gbufA[i, pl.ds(0, 16)] = fz
        gbufA[i, pl.ds(16, 16)] = fz
    for k in range(24):
        pltpu.sync_copy(gbufA, acc.at[pl.ds(row0 + k * 128, 128)])
    pltpu.sync_copy(gbufA.at[pl.ds(0, RPT - 3072)],
                    acc.at[pl.ds(row0 + 3072, RPT - 3072)])
    plsc.subcore_barrier()

    edges_per_tile = E_PAD // NS
    nblocks = edges_per_tile // 1024
    base_e = s * edges_per_tile
    gbufs = (gbufA, gbufB)
    gsems = (gsemA, gsemB)
    ssems = (ssemA, ssemB)

    def didx_at(j):
        return didx.at[pl.ds(j * 128, 128)]

    def gidx_at(j):
        return gidx.at[pl.ds(j * 128, 128)]

    def drain(j):
        pltpu.make_async_copy(
            gbufs[j % 2], acc.at[didx_at(j)], ssems[j % 2]).wait()

    def finish_gather_fire_scatter(j):
        gbuf = gbufs[j % 2]
        pltpu.make_async_copy(h1_hbm.at[gidx_at(j)], gbuf, gsems[j % 2]).wait()
        pltpu.make_async_copy(
            gbuf, acc.at[didx_at(j)], ssems[j % 2]).start(add=True)

    def body(b, carry):
        @pl.when(b > 0)
        def _():
            drain(6)
            drain(7)
        e0 = base_e + b * 1024
        pltpu.sync_copy(src_hbm.at[pl.ds(e0, 1024)], sidx)
        pltpu.sync_copy(dst_hbm.at[pl.ds(e0, 1024)], didx)
        for g in range(64):
            s16 = sidx[pl.ds(g * 16, 16)]
            gidx[pl.ds(g * 16, 16)] = s16 + c * N_NODES
        for j in range(8):
            gbuf, gs = gbufs[j % 2], gsems[j % 2]
            if j >= 2:
                drain(j - 2)
            pltpu.make_async_copy(h1_hbm.at[gidx_at(j)], gbuf, gs).start()
            if j >= 1:
                finish_gather_fire_scatter(j - 1)
        finish_gather_fire_scatter(7)
        return carry

    lax.fori_loop(0, nblocks, body, 0)
    drain(6)
    drain(7)
    plsc.subcore_barrier()

    for k in range(24):
        pltpu.sync_copy(acc.at[pl.ds(row0 + k * 128, 128)], gbufA)
        pltpu.sync_copy(gbufA, out_hbm.at[c, pl.ds(row0 + k * 128, 128)])
    rem = RPT - 3072
    pltpu.sync_copy(acc.at[pl.ds(row0 + 3072, rem)], gbufA.at[pl.ds(0, rem)])
    pltpu.sync_copy(gbufA.at[pl.ds(0, rem)],
                    out_hbm.at[c, pl.ds(row0 + 3072, rem)])


_R = 2000


def _split_acc(a):
    p = a[:, 0:1] + a[:, 1:2]
    m = a[:, 2:3] + a[:, 3:4]
    hf = jnp.concatenate(
        [a[:, 4:5] + a[:, 5:6], a[:, 6:7] + a[:, 7:8]], axis=1)
    return p, m, hf


def _dense1_body(x_ref, acc_ref, Wx_ref, Wf_ref, bf_ref, wvec_ref, Ww_ref,
                 out_ref):
    p, m, hf = _split_acc(acc_ref[...])
    wv = wvec_ref[...]
    u = jnp.sum(jnp.maximum(wv, 0.0) * Ww_ref[...], axis=0, keepdims=True)
    v = jnp.sum(jnp.maximum(-wv, 0.0) * Ww_ref[...], axis=0, keepdims=True)
    h = (jnp.dot(x_ref[...], Wx_ref[...], preferred_element_type=jnp.float32)
         + jnp.dot(hf, Wf_ref[...], preferred_element_type=jnp.float32)
         + p * u + m * v + bf_ref[...])
    h = jnp.maximum(h, 0.0)
    out_ref[0] = h[:, :32]
    out_ref[1] = h[:, 32:]


def _dense2_body(x_ref, acc_ref, hf2_ref, Wx_ref, Wf_ref, bf_ref, wvec_ref,
                 Ww_ref, out_ref):
    p, m, _ = _split_acc(acc_ref[...])
    wv = wvec_ref[...]
    u = jnp.sum(jnp.maximum(wv, 0.0) * Ww_ref[...], axis=0, keepdims=True)
    v = jnp.sum(jnp.maximum(-wv, 0.0) * Ww_ref[...], axis=0, keepdims=True)
    Wf = Wf_ref[...]
    h = (jnp.dot(x_ref[...], Wx_ref[...], preferred_element_type=jnp.float32)
         + jnp.dot(hf2_ref[0], Wf[:32, :], preferred_element_type=jnp.float32)
         + jnp.dot(hf2_ref[1], Wf[32:, :], preferred_element_type=jnp.float32)
         + p * u + m * v + bf_ref[...])
    out_ref[...] = jnp.maximum(h, 0.0)


def _dense1(x, acc, Wx, Wf, bf, wvec, Ww):
    grid = N_NODES // _R
    return pl.pallas_call(
        _dense1_body,
        grid=(grid,),
        in_specs=[
            pl.BlockSpec((_R, 2), lambda i: (i, 0)),
            pl.BlockSpec((_R, 8), lambda i: (i, 0)),
            pl.BlockSpec((2, 64), lambda i: (0, 0)),
            pl.BlockSpec((2, 64), lambda i: (0, 0)),
            pl.BlockSpec((1, 64), lambda i: (0, 0)),
            pl.BlockSpec((64, 1), lambda i: (0, 0)),
            pl.BlockSpec((64, 64), lambda i: (0, 0)),
        ],
        out_specs=pl.BlockSpec((2, _R, 32), lambda i: (0, i, 0)),
        out_shape=jax.ShapeDtypeStruct((2, N_NODES, 32), jnp.float32),
    )(x, acc, Wx, Wf, bf, wvec, Ww)


def _dense2(x, acc, hf2, Wx, Wf, bf, wvec, Ww):
    grid = N_NODES // _R
    return pl.pallas_call(
        _dense2_body,
        grid=(grid,),
        in_specs=[
            pl.BlockSpec((_R, 2), lambda i: (i, 0)),
            pl.BlockSpec((_R, 8), lambda i: (i, 0)),
            pl.BlockSpec((2, _R, 32), lambda i: (0, i, 0)),
            pl.BlockSpec((2, 64), lambda i: (0, 0)),
            pl.BlockSpec((64, 64), lambda i: (0, 0)),
            pl.BlockSpec((1, 64), lambda i: (0, 0)),
            pl.BlockSpec((64, 1), lambda i: (0, 0)),
            pl.BlockSpec((64, 64), lambda i: (0, 0)),
        ],
        out_specs=pl.BlockSpec((_R, 64), lambda i: (i, 0)),
        out_shape=jax.ShapeDtypeStruct((N_NODES, 64), jnp.float32),
    )(x, acc, hf2, Wx, Wf, bf, wvec, Ww)


def kernel(x, edge_index, edge_w, Wx1, Ww1, Wf1, bf1, wvec1,
           Wx2, Ww2, Wf2, bf2, wvec2):
    src = edge_index[0].astype(jnp.int32)
    dst = edge_index[1].astype(jnp.int32)
    pad = E_PAD - E_EDGES
    pad_dst = N_NODES + (jnp.arange(pad, dtype=jnp.int32) % (N_PAD - N_NODES))
    src_p = jnp.concatenate([src, jnp.zeros((pad,), jnp.int32)])
    dst_p = jnp.concatenate([dst, pad_dst])
    w_p = jnp.concatenate(
        [edge_w.astype(jnp.float32), jnp.zeros((pad,), jnp.float32)])
    x_flat = x.reshape(-1)

    accP, accM, accX0, accX1 = (
        a.reshape(NC, N_PAD) for a in _edge_pass1(src_p, dst_p, w_p, x_flat))
    acc8 = jnp.stack(
        [accP[0], accP[1], accM[0], accM[1],
         accX0[0], accX0[1], accX1[0], accX1[1]], axis=1)
    h1 = _dense1(x, acc8, Wx1, Wf1, bf1.reshape(1, 64),
                 wvec1.reshape(64, 1), Ww1)
    hf2 = _edge_pass2(src_p, dst_p, h1.reshape(2 * N_NODES, 32))
    out = _dense2(x, acc8, hf2, Wx2, Wf2, bf2.reshape(1, 64),
                  wvec2.reshape(64, 1), Ww2)
    return out


# speedup vs baseline: 1.3576x; 1.2534x over previous
import functools

import jax
import jax.numpy as jnp
from jax import lax
from jax.experimental import pallas as pl
from jax.experimental.pallas import tpu as pltpu
from jax.experimental.pallas import tpu_sc as plsc

N_NODES = 50000
E_EDGES = 800000
NC, NS = 2, 16
N_PAD = 50048
E_PAD = 802816
ROWS = E_PAD // 128
RPT = N_PAD // NS

_mesh = plsc.VectorSubcoreMesh(
    core_axis_name="c", subcore_axis_name="s", num_cores=NC, num_subcores=NS)
_sc_params = pltpu.CompilerParams(
    needs_layout_passes=False, use_tc_tiling_on_sc=False)


_vec_out = tuple(
    jax.ShapeDtypeStruct((NC * N_PAD,), jnp.float32) for _ in range(4))


@functools.partial(
    pl.kernel,
    out_type=_vec_out,
    mesh=_mesh,
    scratch_types=[
        pltpu.VMEM((2 * N_NODES,), jnp.float32),
        pltpu.VMEM((512,), jnp.int32),
        pltpu.VMEM((512,), jnp.int32),
        pltpu.VMEM((512,), jnp.float32),
        pltpu.VMEM((4, 128), jnp.float32),
        pltpu.VMEM((4, 128), jnp.float32),
        pltpu.VMEM((4, 128), jnp.float32),
        pltpu.VMEM((4, 128), jnp.float32),
        pltpu.VMEM((1024,), jnp.float32),
        pltpu.VMEM((RPT,), jnp.float32),
        pltpu.VMEM_SHARED((N_PAD,), jnp.float32),
        pltpu.VMEM_SHARED((N_PAD,), jnp.float32),
        pltpu.VMEM_SHARED((N_PAD,), jnp.float32),
        pltpu.VMEM_SHARED((N_PAD,), jnp.float32),
        pltpu.SemaphoreType.DMA,
    ],
    compiler_params=_sc_params,
)
def _edge_pass1(src_hbm, dst_hbm, w_hbm, x_hbm,
                outP, outM, outX0, outX1,
                x_v, sidx, didx, wbuf, pbuf, mbuf, x0b, x1b, zv, obuf,
                accP, accM, accX0, accX1, ssem):
    c = lax.axis_index("c")
    s = lax.axis_index("s")
    wid = c * NS + s
    row0 = s * RPT
    pltpu.sync_copy(x_hbm, x_v)
    fz = jnp.zeros((16,), jnp.float32)
    for i in range(64):
        zv[pl.ds(i * 16, 16)] = fz
    for acc in (accP, accM, accX0, accX1):
        for k in range(3):
            pltpu.sync_copy(zv, acc.at[pl.ds(row0 + k * 1024, 1024)])
        pltpu.sync_copy(zv.at[pl.ds(0, RPT - 3072)],
                        acc.at[pl.ds(row0 + 3072, RPT - 3072)])
    plsc.subcore_barrier()

    nblocks = (E_PAD // 32) // 512
    base_e = wid * (E_PAD // 32)

    def scatters(didx_ref):
        return [pltpu.make_async_copy(
                    vb.at[r], acc.at[didx_ref.at[pl.ds(r * 128, 128)]], ssem)
                for r in range(4)
                for vb, acc in ((pbuf, accP), (mbuf, accM),
                                (x0b, accX0), (x1b, accX1))]

    def body(b, carry):
        e0 = base_e + b * 512
        pltpu.sync_copy(src_hbm.at[pl.ds(e0, 512)], sidx)
        pltpu.sync_copy(dst_hbm.at[pl.ds(e0, 512)], didx)
        pltpu.sync_copy(w_hbm.at[pl.ds(e0, 512)], wbuf)
        for r in range(4):
            for g in range(8):
                o = r * 128 + g * 16
                w16 = wbuf[pl.ds(o, 16)]
                s16 = sidx[pl.ds(o, 16)]
                pbuf[r, pl.ds(g * 16, 16)] = jnp.maximum(w16, 0.0)
                mbuf[r, pl.ds(g * 16, 16)] = jnp.maximum(-w16, 0.0)
                x0b[r, pl.ds(g * 16, 16)] = plsc.load_gather(x_v, [s16 * 2])
                x1b[r, pl.ds(g * 16, 16)] = plsc.load_gather(x_v, [s16 * 2 + 1])
        ds = scatters(didx)
        for d in ds:
            d.start(add=True)
        for d in ds:
            d.wait()
        return carry

    lax.fori_loop(0, nblocks, body, 0)
    plsc.subcore_barrier()
    off = pl.multiple_of(c * N_PAD + row0, 8)
    for acc, out in ((accP, outP), (accM, outM), (accX0, outX0), (accX1, outX1)):
        pltpu.sync_copy(acc.at[pl.ds(row0, RPT)], obuf)
        pltpu.sync_copy(obuf, out.at[pl.ds(off, RPT)])


@functools.partial(
    pl.kernel,
    out_type=jax.ShapeDtypeStruct((NC, N_PAD, 32), jnp.float32),
    mesh=_mesh,
    scratch_types=[
        pltpu.VMEM((1024,), jnp.int32),
        pltpu.VMEM((1024,), jnp.int32),
        pltpu.VMEM((1024,), jnp.int32),
        pltpu.VMEM((128, 32), jnp.float32),
        pltpu.VMEM((128, 32), jnp.float32),
        pltpu.VMEM_SHARED((N_PAD, 32), jnp.float32),
        pltpu.SemaphoreType.DMA,
        pltpu.SemaphoreType.DMA,
        pltpu.SemaphoreType.DMA,
        pltpu.SemaphoreType.DMA,
    ],
    compiler_params=_sc_params,
)
def _edge_pass2(src_hbm, dst_hbm, h1_hbm, out_hbm,
                sidx, gidx, didx, gbufA, gbufB, acc,
                gsemA, gsemB, ssemA, ssemB):
    c = lax.axis_index("c")
    s = lax.axis_index("s")
    row0 = s * RPT

    fz = jnp.zeros((16,), jnp.float32)
    for i in range(128):
        gbufA[i, pl.ds(0, 16)] = fz
        gbufA[i, pl.ds(16, 16)] = fz
    for k in range(24):
        pltpu.sync_copy(gbufA, acc.at[pl.ds(row0 + k * 128, 128)])
    pltpu.sync_copy(gbufA.at[pl.ds(0, RPT - 3072)],
                    acc.at[pl.ds(row0 + 3072, RPT - 3072)])
    plsc.subcore_barrier()

    edges_per_tile = E_PAD // NS
    nblocks = edges_per_tile // 1024
    base_e = s * edges_per_tile
    gbufs = (gbufA, gbufB)
    gsems = (gsemA, gsemB)
    ssems = (ssemA, ssemB)

    def didx_at(j):
        return didx.at[pl.ds(j * 128, 128)]

    def gidx_at(j):
        return gidx.at[pl.ds(j * 128, 128)]

    def drain(j):
        pltpu.make_async_copy(
            gbufs[j % 2], acc.at[didx_at(j)], ssems[j % 2]).wait()

    def finish_gather_fire_scatter(j):
        gbuf = gbufs[j % 2]
        pltpu.make_async_copy(h1_hbm.at[gidx_at(j)], gbuf, gsems[j % 2]).wait()
        pltpu.make_async_copy(
            gbuf, acc.at[didx_at(j)], ssems[j % 2]).start(add=True)

    def body(b, carry):
        @pl.when(b > 0)
        def _():
            drain(6)
            drain(7)
        e0 = base_e + b * 1024
        pltpu.sync_copy(src_hbm.at[pl.ds(e0, 1024)], sidx)
        pltpu.sync_copy(dst_hbm.at[pl.ds(e0, 1024)], didx)
        for g in range(64):
            s16 = sidx[pl.ds(g * 16, 16)]
            gidx[pl.ds(g * 16, 16)] = s16 + c * N_NODES
        for j in range(8):
            gbuf, gs = gbufs[j % 2], gsems[j % 2]
            if j >= 2:
                drain(j - 2)
            pltpu.make_async_copy(h1_hbm.at[gidx_at(j)], gbuf, gs).start()
            if j >= 1:
                finish_gather_fire_scatter(j - 1)
        finish_gather_fire_scatter(7)
        return carry

    lax.fori_loop(0, nblocks, body, 0)
    drain(6)
    drain(7)
    plsc.subcore_barrier()

    for k in range(24):
        pltpu.sync_copy(acc.at[pl.ds(row0 + k * 128, 128)], gbufA)
        pltpu.sync_copy(gbufA, out_hbm.at[c, pl.ds(row0 + k * 128, 128)])
    rem = RPT - 3072
    pltpu.sync_copy(acc.at[pl.ds(row0 + 3072, rem)], gbufA.at[pl.ds(0, rem)])
    pltpu.sync_copy(gbufA.at[pl.ds(0, rem)],
                    out_hbm.at[c, pl.ds(row0 + 3072, rem)])


_R = 5000


def _split_acc(a):
    p = a[:, 0:1] + a[:, 1:2]
    m = a[:, 2:3] + a[:, 3:4]
    hf = jnp.concatenate(
        [a[:, 4:5] + a[:, 5:6], a[:, 6:7] + a[:, 7:8]], axis=1)
    return p, m, hf


def _dense1_body(x_ref, acc_ref, Wx_ref, Wf_ref, bf_ref, wvec_ref, Ww_ref,
                 out_ref):
    p, m, hf = _split_acc(acc_ref[...])
    wv = wvec_ref[...]
    u = jnp.sum(jnp.maximum(wv, 0.0) * Ww_ref[...], axis=0, keepdims=True)
    v = jnp.sum(jnp.maximum(-wv, 0.0) * Ww_ref[...], axis=0, keepdims=True)
    h = (jnp.dot(x_ref[...], Wx_ref[...], preferred_element_type=jnp.float32)
         + jnp.dot(hf, Wf_ref[...], preferred_element_type=jnp.float32)
         + p * u + m * v + bf_ref[...])
    h = jnp.maximum(h, 0.0)
    out_ref[0] = h[:, :32]
    out_ref[1] = h[:, 32:]


def _dense2_body(x_ref, acc_ref, hf2_ref, Wx_ref, Wf_ref, bf_ref, wvec_ref,
                 Ww_ref, out_ref):
    p, m, _ = _split_acc(acc_ref[...])
    wv = wvec_ref[...]
    u = jnp.sum(jnp.maximum(wv, 0.0) * Ww_ref[...], axis=0, keepdims=True)
    v = jnp.sum(jnp.maximum(-wv, 0.0) * Ww_ref[...], axis=0, keepdims=True)
    Wf = Wf_ref[...]
    h = (jnp.dot(x_ref[...], Wx_ref[...], preferred_element_type=jnp.float32)
         + jnp.dot(hf2_ref[0], Wf[:32, :], preferred_element_type=jnp.float32)
         + jnp.dot(hf2_ref[1], Wf[32:, :], preferred_element_type=jnp.float32)
         + p * u + m * v + bf_ref[...])
    out_ref[...] = jnp.maximum(h, 0.0)


def _dense1(x, acc, Wx, Wf, bf, wvec, Ww):
    grid = N_NODES // _R
    return pl.pallas_call(
        _dense1_body,
        grid=(grid,),
        in_specs=[
            pl.BlockSpec((_R, 2), lambda i: (i, 0)),
            pl.BlockSpec((_R, 8), lambda i: (i, 0)),
            pl.BlockSpec((2, 64), lambda i: (0, 0)),
            pl.BlockSpec((2, 64), lambda i: (0, 0)),
            pl.BlockSpec((1, 64), lambda i: (0, 0)),
            pl.BlockSpec((64, 1), lambda i: (0, 0)),
            pl.BlockSpec((64, 64), lambda i: (0, 0)),
        ],
        out_specs=pl.BlockSpec((2, _R, 32), lambda i: (0, i, 0)),
        out_shape=jax.ShapeDtypeStruct((2, N_NODES, 32), jnp.float32),
    )(x, acc, Wx, Wf, bf, wvec, Ww)


def _dense2(x, acc, hf2, Wx, Wf, bf, wvec, Ww):
    grid = N_NODES // _R
    return pl.pallas_call(
        _dense2_body,
        grid=(grid,),
        in_specs=[
            pl.BlockSpec((_R, 2), lambda i: (i, 0)),
            pl.BlockSpec((_R, 8), lambda i: (i, 0)),
            pl.BlockSpec((2, _R, 32), lambda i: (0, i, 0)),
            pl.BlockSpec((2, 64), lambda i: (0, 0)),
            pl.BlockSpec((64, 64), lambda i: (0, 0)),
            pl.BlockSpec((1, 64), lambda i: (0, 0)),
            pl.BlockSpec((64, 1), lambda i: (0, 0)),
            pl.BlockSpec((64, 64), lambda i: (0, 0)),
        ],
        out_specs=pl.BlockSpec((_R, 64), lambda i: (i, 0)),
        out_shape=jax.ShapeDtypeStruct((N_NODES, 64), jnp.float32),
    )(x, acc, hf2, Wx, Wf, bf, wvec, Ww)


def kernel(x, edge_index, edge_w, Wx1, Ww1, Wf1, bf1, wvec1,
           Wx2, Ww2, Wf2, bf2, wvec2):
    src = edge_index[0].astype(jnp.int32)
    dst = edge_index[1].astype(jnp.int32)
    pad = E_PAD - E_EDGES
    pad_dst = N_NODES + (jnp.arange(pad, dtype=jnp.int32) % (N_PAD - N_NODES))
    src_p = jnp.concatenate([src, jnp.zeros((pad,), jnp.int32)])
    dst_p = jnp.concatenate([dst, pad_dst])
    w_p = jnp.concatenate(
        [edge_w.astype(jnp.float32), jnp.zeros((pad,), jnp.float32)])
    x_flat = x.reshape(-1)

    accP, accM, accX0, accX1 = (
        a.reshape(NC, N_PAD) for a in _edge_pass1(src_p, dst_p, w_p, x_flat))
    acc8 = jnp.stack(
        [accP[0], accP[1], accM[0], accM[1],
         accX0[0], accX0[1], accX1[0], accX1[1]], axis=1)
    h1 = _dense1(x, acc8, Wx1, Wf1, bf1.reshape(1, 64),
                 wvec1.reshape(64, 1), Ww1)
    hf2 = _edge_pass2(src_p, dst_p, h1.reshape(2 * N_NODES, 32))
    out = _dense2(x, acc8, hf2, Wx2, Wf2, bf2.reshape(1, 64),
                  wvec2.reshape(64, 1), Ww2)
    return out
